# baseline (speedup 1.0000x reference)
import jax
import jax.numpy as jnp
from jax import lax
from jax.experimental import pallas as pl

_N_TOKENS = 32768
_EMBED_DIM = 768
_NUM_EXPERTS = 8
_TOP_K = 2
_BLOCK = 2048


def _router_body(x_ref, wg_ref, wn_ref, bg_ref, bn_ref, eps_ref,
                 gate_ref, noisy_ref, w1_ref, w2_ref, i1_ref, i2_ref):
    x = x_ref[...]
    g = jnp.dot(x, wg_ref[...], preferred_element_type=jnp.float32) + bg_ref[...]
    nl = jnp.dot(x, wn_ref[...], preferred_element_type=jnp.float32) + bn_ref[...]
    sp = jnp.maximum(nl, 0.0) + jnp.log1p(jnp.exp(-jnp.abs(nl)))
    noisy = g + eps_ref[...] * sp
    gate_ref[...] = g
    noisy_ref[...] = noisy

    iota = lax.broadcasted_iota(jnp.int32, noisy.shape, 1)
    m1 = jnp.max(noisy, axis=1, keepdims=True)
    i1 = jnp.min(jnp.where(noisy == m1, iota, _NUM_EXPERTS), axis=1, keepdims=True)
    masked = jnp.where(iota == i1, -jnp.inf, noisy)
    m2 = jnp.max(masked, axis=1, keepdims=True)
    i2 = jnp.min(jnp.where(masked == m2, iota, _NUM_EXPERTS), axis=1, keepdims=True)
    e2 = jnp.exp(m2 - m1)
    denom = 1.0 + e2
    w1_ref[...] = 1.0 / denom
    w2_ref[...] = e2 / denom
    i1_ref[...] = i1
    i2_ref[...] = i2


def kernel(hidden_states, Wg, bg, Wn, bn):
    n, d = hidden_states.shape
    e = Wg.shape[1]
    eps = jax.random.normal(jax.random.key(42), (n, e), dtype=jnp.float32)
    bg2 = bg.reshape(1, e)
    bn2 = bn.reshape(1, e)
    grid = (n // _BLOCK,)
    outs = pl.pallas_call(
        _router_body,
        grid=grid,
        in_specs=[
            pl.BlockSpec((_BLOCK, d), lambda i: (i, 0)),
            pl.BlockSpec((d, e), lambda i: (0, 0)),
            pl.BlockSpec((d, e), lambda i: (0, 0)),
            pl.BlockSpec((1, e), lambda i: (0, 0)),
            pl.BlockSpec((1, e), lambda i: (0, 0)),
            pl.BlockSpec((_BLOCK, e), lambda i: (i, 0)),
        ],
        out_specs=[
            pl.BlockSpec((_BLOCK, e), lambda i: (i, 0)),
            pl.BlockSpec((_BLOCK, e), lambda i: (i, 0)),
            pl.BlockSpec((_BLOCK, 1), lambda i: (i, 0)),
            pl.BlockSpec((_BLOCK, 1), lambda i: (i, 0)),
            pl.BlockSpec((_BLOCK, 1), lambda i: (i, 0)),
            pl.BlockSpec((_BLOCK, 1), lambda i: (i, 0)),
        ],
        out_shape=[
            jax.ShapeDtypeStruct((n, e), jnp.float32),
            jax.ShapeDtypeStruct((n, e), jnp.float32),
            jax.ShapeDtypeStruct((n, 1), jnp.float32),
            jax.ShapeDtypeStruct((n, 1), jnp.float32),
            jax.ShapeDtypeStruct((n, 1), jnp.int32),
            jax.ShapeDtypeStruct((n, 1), jnp.int32),
        ],
    )(hidden_states, Wg, Wn, bg2, bn2, eps)
    gate_logits, noisy_logits, w1, w2, i1, i2 = outs
    routing_weights = jnp.concatenate([w1, w2], axis=1)
    selected_experts = jnp.concatenate([i1, i2], axis=1)
    return (routing_weights, selected_experts, noisy_logits, gate_logits)


# speedup vs baseline: 3.0199x; 3.0199x over previous
import jax
import jax.numpy as jnp
from jax import lax
from jax.experimental import pallas as pl

_N_TOKENS = 32768
_EMBED_DIM = 768
_NUM_EXPERTS = 8
_TOP_K = 2
_BLOCK = 2048


def _router_body(x_ref, w_ref, b_ref, epsT_ref,
                 gate_ref, noisy_ref, w1_ref, w2_ref, i1_ref, i2_ref):
    x = x_ref[...]
    e = _NUM_EXPERTS
    logits = jnp.dot(x, w_ref[...], preferred_element_type=jnp.float32) + b_ref[...]
    gate_ref[...] = logits[:, :e]

    lt = logits.T
    g_t = lt[:e, :]
    n_t = lt[e:, :]
    sp = jnp.maximum(n_t, 0.0) + jnp.log1p(jnp.exp(-jnp.abs(n_t)))
    noisy_t = g_t + epsT_ref[...] * sp
    noisy_ref[...] = noisy_t.T

    iota = lax.broadcasted_iota(jnp.int32, noisy_t.shape, 0)
    m1 = jnp.max(noisy_t, axis=0, keepdims=True)
    i1 = jnp.min(jnp.where(noisy_t == m1, iota, e), axis=0, keepdims=True)
    masked = jnp.where(iota == i1, -jnp.inf, noisy_t)
    m2 = jnp.max(masked, axis=0, keepdims=True)
    i2 = jnp.min(jnp.where(masked == m2, iota, e), axis=0, keepdims=True)
    e2 = jnp.exp(m2 - m1)
    denom = 1.0 + e2
    w1_ref[...] = 1.0 / denom
    w2_ref[...] = e2 / denom
    i1_ref[...] = i1
    i2_ref[...] = i2


def kernel(hidden_states, Wg, bg, Wn, bn):
    n, d = hidden_states.shape
    e = Wg.shape[1]
    eps = jax.random.normal(jax.random.key(42), (n, e), dtype=jnp.float32)
    epsT = eps.T
    w16 = jnp.concatenate([Wg, Wn], axis=1)
    b16 = jnp.concatenate([bg, bn]).reshape(1, 2 * e)
    grid = (n // _BLOCK,)
    outs = pl.pallas_call(
        _router_body,
        grid=grid,
        in_specs=[
            pl.BlockSpec((_BLOCK, d), lambda i: (i, 0)),
            pl.BlockSpec((d, 2 * e), lambda i: (0, 0)),
            pl.BlockSpec((1, 2 * e), lambda i: (0, 0)),
            pl.BlockSpec((e, _BLOCK), lambda i: (0, i)),
        ],
        out_specs=[
            pl.BlockSpec((_BLOCK, e), lambda i: (i, 0)),
            pl.BlockSpec((_BLOCK, e), lambda i: (i, 0)),
            pl.BlockSpec((1, _BLOCK), lambda i: (0, i)),
            pl.BlockSpec((1, _BLOCK), lambda i: (0, i)),
            pl.BlockSpec((1, _BLOCK), lambda i: (0, i)),
            pl.BlockSpec((1, _BLOCK), lambda i: (0, i)),
        ],
        out_shape=[
            jax.ShapeDtypeStruct((n, e), jnp.float32),
            jax.ShapeDtypeStruct((n, e), jnp.float32),
            jax.ShapeDtypeStruct((1, n), jnp.float32),
            jax.ShapeDtypeStruct((1, n), jnp.float32),
            jax.ShapeDtypeStruct((1, n), jnp.int32),
            jax.ShapeDtypeStruct((1, n), jnp.int32),
        ],
    )(hidden_states, w16, b16, epsT)
    gate_logits, noisy_logits, w1, w2, i1, i2 = outs
    routing_weights = jnp.concatenate([w1, w2], axis=0).T
    selected_experts = jnp.concatenate([i1, i2], axis=0).T
    return (routing_weights, selected_experts, noisy_logits, gate_logits)


# speedup vs baseline: 3.0459x; 1.0086x over previous
import jax
import jax.numpy as jnp
from jax import lax
from jax.experimental import pallas as pl

_N_TOKENS = 32768
_EMBED_DIM = 768
_NUM_EXPERTS = 8
_TOP_K = 2
_BLOCK = 4096


def _router_body(x_ref, w_ref, b_ref, epsT_ref,
                 gate_ref, noisy_ref, w1_ref, w2_ref, i1_ref, i2_ref):
    x = x_ref[...]
    e = _NUM_EXPERTS
    logits = jnp.dot(x, w_ref[...], preferred_element_type=jnp.float32) + b_ref[...]
    gate_ref[...] = logits[:, :e]

    lt = logits.T
    g_t = lt[:e, :]
    n_t = lt[e:, :]
    sp = jnp.maximum(n_t, 0.0) + jnp.log1p(jnp.exp(-jnp.abs(n_t)))
    noisy_t = g_t + epsT_ref[...] * sp
    noisy_ref[...] = noisy_t.T

    iota = lax.broadcasted_iota(jnp.int32, noisy_t.shape, 0)
    m1 = jnp.max(noisy_t, axis=0, keepdims=True)
    i1 = jnp.min(jnp.where(noisy_t == m1, iota, e), axis=0, keepdims=True)
    masked = jnp.where(iota == i1, -jnp.inf, noisy_t)
    m2 = jnp.max(masked, axis=0, keepdims=True)
    i2 = jnp.min(jnp.where(masked == m2, iota, e), axis=0, keepdims=True)
    e2 = jnp.exp(m2 - m1)
    denom = 1.0 + e2
    w1_ref[...] = 1.0 / denom
    w2_ref[...] = e2 / denom
    i1_ref[...] = i1
    i2_ref[...] = i2


def kernel(hidden_states, Wg, bg, Wn, bn):
    n, d = hidden_states.shape
    e = Wg.shape[1]
    eps = jax.random.normal(jax.random.key(42), (n, e), dtype=jnp.float32)
    epsT = eps.T
    w16 = jnp.concatenate([Wg, Wn], axis=1)
    b16 = jnp.concatenate([bg, bn]).reshape(1, 2 * e)
    grid = (n // _BLOCK,)
    outs = pl.pallas_call(
        _router_body,
        grid=grid,
        in_specs=[
            pl.BlockSpec((_BLOCK, d), lambda i: (i, 0)),
            pl.BlockSpec((d, 2 * e), lambda i: (0, 0)),
            pl.BlockSpec((1, 2 * e), lambda i: (0, 0)),
            pl.BlockSpec((e, _BLOCK), lambda i: (0, i)),
        ],
        out_specs=[
            pl.BlockSpec((_BLOCK, e), lambda i: (i, 0)),
            pl.BlockSpec((_BLOCK, e), lambda i: (i, 0)),
            pl.BlockSpec((1, _BLOCK), lambda i: (0, i)),
            pl.BlockSpec((1, _BLOCK), lambda i: (0, i)),
            pl.BlockSpec((1, _BLOCK), lambda i: (0, i)),
            pl.BlockSpec((1, _BLOCK), lambda i: (0, i)),
        ],
        out_shape=[
            jax.ShapeDtypeStruct((n, e), jnp.float32),
            jax.ShapeDtypeStruct((n, e), jnp.float32),
            jax.ShapeDtypeStruct((1, n), jnp.float32),
            jax.ShapeDtypeStruct((1, n), jnp.float32),
            jax.ShapeDtypeStruct((1, n), jnp.int32),
            jax.ShapeDtypeStruct((1, n), jnp.int32),
        ],
    )(hidden_states, w16, b16, epsT)
    gate_logits, noisy_logits, w1, w2, i1, i2 = outs
    routing_weights = jnp.concatenate([w1, w2], axis=0).T
    selected_experts = jnp.concatenate([i1, i2], axis=0).T
    return (routing_weights, selected_experts, noisy_logits, gate_logits)


# speedup vs baseline: 3.1196x; 1.0242x over previous
import jax
import jax.numpy as jnp
from jax import lax
from jax.experimental import pallas as pl

_N_TOKENS = 32768
_EMBED_DIM = 768
_NUM_EXPERTS = 8
_TOP_K = 2
_BLOCK = 4096


def _router_body(xa_ref, xb_ref, wa_ref, wb_ref, b_ref, epsT_ref,
                 gate_ref, noisy_ref, w1_ref, w2_ref, i1_ref, i2_ref):
    e = _NUM_EXPERTS
    logits = (jnp.dot(xa_ref[...], wa_ref[...], preferred_element_type=jnp.float32)
              + jnp.dot(xb_ref[...], wb_ref[...], preferred_element_type=jnp.float32)
              + b_ref[...])
    gate_ref[...] = logits[:, :e]

    lt = logits.T
    g_t = lt[:e, :]
    n_t = lt[e:, :]
    sp = jnp.maximum(n_t, 0.0) + jnp.log1p(jnp.exp(-jnp.abs(n_t)))
    noisy_t = g_t + epsT_ref[...] * sp
    noisy_ref[...] = noisy_t.T

    iota = lax.broadcasted_iota(jnp.int32, noisy_t.shape, 0)
    m1 = jnp.max(noisy_t, axis=0, keepdims=True)
    i1 = jnp.min(jnp.where(noisy_t == m1, iota, e), axis=0, keepdims=True)
    masked = jnp.where(iota == i1, -jnp.inf, noisy_t)
    m2 = jnp.max(masked, axis=0, keepdims=True)
    i2 = jnp.min(jnp.where(masked == m2, iota, e), axis=0, keepdims=True)
    e2 = jnp.exp(m2 - m1)
    denom = 1.0 + e2
    w1_ref[...] = 1.0 / denom
    w2_ref[...] = e2 / denom
    i1_ref[...] = i1
    i2_ref[...] = i2


def kernel(hidden_states, Wg, bg, Wn, bn):
    n, d = hidden_states.shape
    e = Wg.shape[1]
    eps = jax.random.normal(jax.random.key(42), (n, e), dtype=jnp.float32)
    epsT = eps.T
    w16 = jnp.concatenate([Wg, Wn], axis=1)
    b16 = jnp.concatenate([bg, bn]).reshape(1, 2 * e)
    grid = (n // _BLOCK,)
    outs = pl.pallas_call(
        _router_body,
        grid=grid,
        in_specs=[
            pl.BlockSpec((_BLOCK, d // 2), lambda i: (i, 0)),
            pl.BlockSpec((_BLOCK, d // 2), lambda i: (i, 1)),
            pl.BlockSpec((d // 2, 2 * e), lambda i: (0, 0)),
            pl.BlockSpec((d // 2, 2 * e), lambda i: (1, 0)),
            pl.BlockSpec((1, 2 * e), lambda i: (0, 0)),
            pl.BlockSpec((e, _BLOCK), lambda i: (0, i)),
        ],
        out_specs=[
            pl.BlockSpec((_BLOCK, e), lambda i: (i, 0)),
            pl.BlockSpec((_BLOCK, e), lambda i: (i, 0)),
            pl.BlockSpec((1, _BLOCK), lambda i: (0, i)),
            pl.BlockSpec((1, _BLOCK), lambda i: (0, i)),
            pl.BlockSpec((1, _BLOCK), lambda i: (0, i)),
            pl.BlockSpec((1, _BLOCK), lambda i: (0, i)),
        ],
        out_shape=[
            jax.ShapeDtypeStruct((n, e), jnp.float32),
            jax.ShapeDtypeStruct((n, e), jnp.float32),
            jax.ShapeDtypeStruct((1, n), jnp.float32),
            jax.ShapeDtypeStruct((1, n), jnp.float32),
            jax.ShapeDtypeStruct((1, n), jnp.int32),
            jax.ShapeDtypeStruct((1, n), jnp.int32),
        ],
    )(hidden_states, hidden_states, w16, w16, b16, epsT)
    gate_logits, noisy_logits, w1, w2, i1, i2 = outs
    routing_weights = jnp.concatenate([w1, w2], axis=0).T
    selected_experts = jnp.concatenate([i1, i2], axis=0).T
    return (routing_weights, selected_experts, noisy_logits, gate_logits)
